# baseline (speedup 1.0000x reference)
import functools

import jax
import jax.numpy as jnp
from jax.experimental import pallas as pl
from jax.experimental.pallas import tpu as pltpu


def _pick_tile(n, unit, cap):
    best = None
    t = unit
    limit = min(n, cap)
    while t <= limit:
        if n % t == 0:
            best = t
        t += unit
    return best if best is not None else n


def _fused_bn_fc_kernel(x_ref, g_ref, bt_ref, w_ref, bias_ref,
                        o_ref,
                        wb_ref, sum_ref, sumsq_ref, p_ref,
                        *, inv_n, eps, tb):
    j = pl.program_id(0)
    K, F = w_ref.shape
    C = g_ref.shape[1]
    HW = F // C

    @pl.when(j == 0)
    def _():
        sum_ref[...] = jnp.zeros_like(sum_ref)
        sumsq_ref[...] = jnp.zeros_like(sumsq_ref)
        wb_ref[...] = w_ref[...].astype(jnp.bfloat16)

    xb = x_ref[...]
    xf = xb.astype(jnp.float32)
    xg = xf.reshape(tb // 8, 8, F)
    sum_ref[...] += jnp.sum(xg, axis=0)
    sumsq_ref[...] += jnp.sum(xg * xg, axis=0)

    for c in range(C):
        pc = jax.lax.dot_general(
            xb[:, c * HW:(c + 1) * HW], wb_ref[:, c * HW:(c + 1) * HW],
            dimension_numbers=(((1,), (1,)), ((), ())),
            preferred_element_type=jnp.float32)
        p_ref[c, pl.ds(j * tb, tb), :] = pc

    @pl.when(j == pl.num_programs(0) - 1)
    def _():
        tot = jnp.sum(sum_ref[...], axis=0, keepdims=True)
        totsq = jnp.sum(sumsq_ref[...], axis=0, keepdims=True)
        sums = tot.reshape(C, HW)
        sqs = totsq.reshape(C, HW)
        mean = jnp.sum(sums, axis=1, keepdims=True) * inv_n
        var = jnp.sum(sqs, axis=1, keepdims=True) * inv_n - mean * mean
        var = jnp.maximum(var, 0.0)
        g_col = jnp.transpose(g_ref[...])
        bt_col = jnp.transpose(bt_ref[...])
        s = g_col * jax.lax.rsqrt(var + eps)
        t = bt_col - mean * s

        ones_hw = jnp.ones((1, HW), dtype=jnp.bfloat16)
        cst = bias_ref[...]
        for cc in range(C):
            wsum_c = jax.lax.dot_general(
                ones_hw, wb_ref[:, cc * HW:(cc + 1) * HW],
                dimension_numbers=(((1,), (1,)), ((), ())),
                preferred_element_type=jnp.float32)
            cst = cst + t[cc:cc + 1, :] * wsum_c

        acc = jnp.zeros(o_ref.shape, dtype=jnp.float32)
        for cc in range(C):
            acc = acc + p_ref[cc] * s[cc:cc + 1, :]
        o_ref[...] = acc + cst


def kernel(x, gamma, beta, weight, bias):
    B, C, H, W = x.shape
    HW = H * W
    F = C * HW
    K = weight.shape[0]

    x2b = x.astype(jnp.bfloat16).reshape(B, F)

    tb = _pick_tile(B, 8, max(8, min(256, B // 4)))
    grid = (B // tb,)

    out = pl.pallas_call(
        functools.partial(_fused_bn_fc_kernel,
                          inv_n=1.0 / float(B * HW), eps=1e-5, tb=tb),
        out_shape=jax.ShapeDtypeStruct((B, K), jnp.float32),
        grid=grid,
        in_specs=[pl.BlockSpec((tb, F), lambda j: (j, 0)),
                  pl.BlockSpec((1, C), lambda j: (0, 0)),
                  pl.BlockSpec((1, C), lambda j: (0, 0)),
                  pl.BlockSpec((K, F), lambda j: (0, 0)),
                  pl.BlockSpec((1, K), lambda j: (0, 0))],
        out_specs=pl.BlockSpec((B, K), lambda j: (0, 0)),
        scratch_shapes=[pltpu.VMEM((K, F), jnp.bfloat16),
                        pltpu.VMEM((8, F), jnp.float32),
                        pltpu.VMEM((8, F), jnp.float32),
                        pltpu.VMEM((C, B, K), jnp.float32)],
        compiler_params=pltpu.CompilerParams(
            dimension_semantics=("arbitrary",),
            vmem_limit_bytes=56 * 1024 * 1024),
    )(x2b, gamma.reshape(1, C), beta.reshape(1, C), weight, bias.reshape(1, K))
    return out


# speedup vs baseline: 1.1340x; 1.1340x over previous
import functools

import jax
import jax.numpy as jnp
from jax.experimental import pallas as pl
from jax.experimental.pallas import tpu as pltpu


def _pick_tile(n, unit, cap):
    best = None
    t = unit
    limit = min(n, cap)
    while t <= limit:
        if n % t == 0:
            best = t
        t += unit
    return best if best is not None else n


def _fused_bn_fc_kernel(x_ref, g_ref, bt_ref, w_ref, bias_ref,
                        o_ref,
                        wb_ref, sum_ref, sumsq_ref, p_ref,
                        *, inv_n, eps, tb):
    j = pl.program_id(0)
    K, F = w_ref.shape
    C = g_ref.shape[1]
    HW = F // C

    @pl.when(j == 0)
    def _():
        sum_ref[...] = jnp.zeros_like(sum_ref)
        sumsq_ref[...] = jnp.zeros_like(sumsq_ref)
        wb_ref[...] = w_ref[...].astype(jnp.bfloat16)

    xf = x_ref[...]
    xg = xf.reshape(tb // 8, 8, F)
    sum_ref[...] += jnp.sum(xg, axis=0)
    sumsq_ref[...] += jnp.sum(xg * xg, axis=0)

    xb = xf.astype(jnp.bfloat16)
    for c in range(C):
        pc = jax.lax.dot_general(
            xb[:, c * HW:(c + 1) * HW], wb_ref[:, c * HW:(c + 1) * HW],
            dimension_numbers=(((1,), (1,)), ((), ())),
            preferred_element_type=jnp.float32)
        p_ref[c, pl.ds(j * tb, tb), :] = pc

    @pl.when(j == pl.num_programs(0) - 1)
    def _():
        tot = jnp.sum(sum_ref[...], axis=0, keepdims=True)
        totsq = jnp.sum(sumsq_ref[...], axis=0, keepdims=True)
        sums = tot.reshape(C, HW)
        sqs = totsq.reshape(C, HW)
        mean = jnp.sum(sums, axis=1, keepdims=True) * inv_n
        var = jnp.sum(sqs, axis=1, keepdims=True) * inv_n - mean * mean
        var = jnp.maximum(var, 0.0)
        g_col = jnp.transpose(g_ref[...])
        bt_col = jnp.transpose(bt_ref[...])
        s = g_col * jax.lax.rsqrt(var + eps)
        t = bt_col - mean * s

        ones_hw = jnp.ones((1, HW), dtype=jnp.bfloat16)
        cst = bias_ref[...]
        for cc in range(C):
            wsum_c = jax.lax.dot_general(
                ones_hw, wb_ref[:, cc * HW:(cc + 1) * HW],
                dimension_numbers=(((1,), (1,)), ((), ())),
                preferred_element_type=jnp.float32)
            cst = cst + t[cc:cc + 1, :] * wsum_c

        acc = jnp.zeros(o_ref.shape, dtype=jnp.float32)
        for cc in range(C):
            acc = acc + p_ref[cc] * s[cc:cc + 1, :]
        o_ref[...] = acc + cst


def kernel(x, gamma, beta, weight, bias):
    B, C, H, W = x.shape
    HW = H * W
    F = C * HW
    K = weight.shape[0]

    x2 = x.reshape(B, F)

    tb = _pick_tile(B, 8, max(8, min(256, B // 4)))
    grid = (B // tb,)

    out = pl.pallas_call(
        functools.partial(_fused_bn_fc_kernel,
                          inv_n=1.0 / float(B * HW), eps=1e-5, tb=tb),
        out_shape=jax.ShapeDtypeStruct((B, K), jnp.float32),
        grid=grid,
        in_specs=[pl.BlockSpec((tb, F), lambda j: (j, 0)),
                  pl.BlockSpec((1, C), lambda j: (0, 0)),
                  pl.BlockSpec((1, C), lambda j: (0, 0)),
                  pl.BlockSpec((K, F), lambda j: (0, 0)),
                  pl.BlockSpec((1, K), lambda j: (0, 0))],
        out_specs=pl.BlockSpec((B, K), lambda j: (0, 0)),
        scratch_shapes=[pltpu.VMEM((K, F), jnp.bfloat16),
                        pltpu.VMEM((8, F), jnp.float32),
                        pltpu.VMEM((8, F), jnp.float32),
                        pltpu.VMEM((C, B, K), jnp.float32)],
        compiler_params=pltpu.CompilerParams(
            dimension_semantics=("arbitrary",),
            vmem_limit_bytes=56 * 1024 * 1024),
    )(x2, gamma.reshape(1, C), beta.reshape(1, C), weight, bias.reshape(1, K))
    return out


# speedup vs baseline: 2.3512x; 2.0734x over previous
import functools

import jax
import jax.numpy as jnp
from jax.experimental import pallas as pl
from jax.experimental.pallas import tpu as pltpu


def _fused_bn_fc_kernel(xt_ref, g_ref, bt_ref, w_ref, bias_ref,
                        o_ref,
                        wb_ref, sum_ref, sumsq_ref, p_ref,
                        *, inv_n, eps):
    c = pl.program_id(0)
    K, F = w_ref.shape
    C = g_ref.shape[1]
    HW = F // C

    @pl.when(c == 0)
    def _():
        for cc in range(C):
            wb_ref[cc] = w_ref[:, cc * HW:(cc + 1) * HW].astype(jnp.bfloat16)

    xf = xt_ref[...]
    sum_ref[c] = jnp.sum(xf, axis=0, keepdims=True)
    sumsq_ref[c] = jnp.sum(xf * xf, axis=0, keepdims=True)

    xb = xf.astype(jnp.bfloat16)
    pc = jax.lax.dot_general(
        xb, wb_ref[c],
        dimension_numbers=(((0,), (1,)), ((), ())),
        preferred_element_type=jnp.float32)
    p_ref[c] = pc

    @pl.when(c == pl.num_programs(0) - 1)
    def _():
        B = o_ref.shape[0]
        sums = sum_ref[...].reshape(C, B)
        sqs = sumsq_ref[...].reshape(C, B)
        mean = jnp.sum(sums, axis=1, keepdims=True) * inv_n
        var = jnp.sum(sqs, axis=1, keepdims=True) * inv_n - mean * mean
        var = jnp.maximum(var, 0.0)
        g_col = jnp.transpose(g_ref[...])
        bt_col = jnp.transpose(bt_ref[...])
        s = g_col * jax.lax.rsqrt(var + eps)
        t = bt_col - mean * s

        ones_hw = jnp.ones((1, HW), dtype=jnp.bfloat16)
        cst = bias_ref[...]
        for cc in range(C):
            wsum_c = jax.lax.dot_general(
                ones_hw, wb_ref[cc],
                dimension_numbers=(((1,), (1,)), ((), ())),
                preferred_element_type=jnp.float32)
            cst = cst + t[cc:cc + 1, :] * wsum_c

        acc = jnp.zeros(o_ref.shape, dtype=jnp.float32)
        for cc in range(C):
            acc = acc + p_ref[cc] * s[cc:cc + 1, :]
        o_ref[...] = acc + cst


def kernel(x, gamma, beta, weight, bias):
    B, C, H, W = x.shape
    HW = H * W
    F = C * HW
    K = weight.shape[0]

    xt = x.reshape(B, F).T

    out = pl.pallas_call(
        functools.partial(_fused_bn_fc_kernel,
                          inv_n=1.0 / float(B * HW), eps=1e-5),
        out_shape=jax.ShapeDtypeStruct((B, K), jnp.float32),
        grid=(C,),
        in_specs=[pl.BlockSpec((HW, B), lambda c: (c, 0)),
                  pl.BlockSpec((1, C), lambda c: (0, 0)),
                  pl.BlockSpec((1, C), lambda c: (0, 0)),
                  pl.BlockSpec((K, F), lambda c: (0, 0)),
                  pl.BlockSpec((1, K), lambda c: (0, 0))],
        out_specs=pl.BlockSpec((B, K), lambda c: (0, 0)),
        scratch_shapes=[pltpu.VMEM((C, K, HW), jnp.bfloat16),
                        pltpu.VMEM((C, 1, B), jnp.float32),
                        pltpu.VMEM((C, 1, B), jnp.float32),
                        pltpu.VMEM((C, B, K), jnp.float32)],
        compiler_params=pltpu.CompilerParams(
            dimension_semantics=("arbitrary",),
            vmem_limit_bytes=56 * 1024 * 1024),
    )(xt, gamma.reshape(1, C), beta.reshape(1, C), weight, bias.reshape(1, K))
    return out


# speedup vs baseline: 2.9376x; 1.2494x over previous
import functools

import jax
import jax.numpy as jnp
from jax.experimental import pallas as pl
from jax.experimental.pallas import tpu as pltpu


def _fused_bn_fc_kernel(xt_ref, g_ref, bt_ref, w_ref, bias_ref,
                        o_ref,
                        wb_ref, sum_ref, sumsq_ref, p_ref,
                        *, inv_n, eps, cpg):
    j = pl.program_id(0)
    K, F = w_ref.shape
    C = g_ref.shape[1]
    HW = F // C

    @pl.when(j == 0)
    def _():
        for cc in range(C):
            wb_ref[cc] = w_ref[:, cc * HW:(cc + 1) * HW].astype(jnp.bfloat16)

    xf = xt_ref[...]
    xb = xf.astype(jnp.bfloat16)
    for i in range(cpg):
        c = cpg * j + i
        xfi = xf[i * HW:(i + 1) * HW]
        sum_ref[c] = jnp.sum(xfi, axis=0, keepdims=True)
        sumsq_ref[c] = jnp.sum(xfi * xfi, axis=0, keepdims=True)
        pc = jax.lax.dot_general(
            xb[i * HW:(i + 1) * HW], wb_ref[c],
            dimension_numbers=(((0,), (1,)), ((), ())),
            preferred_element_type=jnp.float32)
        p_ref[c] = pc

    @pl.when(j == pl.num_programs(0) - 1)
    def _():
        B = o_ref.shape[0]
        sums = sum_ref[...].reshape(C, B)
        sqs = sumsq_ref[...].reshape(C, B)
        mean = jnp.sum(sums, axis=1, keepdims=True) * inv_n
        var = jnp.sum(sqs, axis=1, keepdims=True) * inv_n - mean * mean
        var = jnp.maximum(var, 0.0)
        g_col = jnp.transpose(g_ref[...])
        bt_col = jnp.transpose(bt_ref[...])
        s = g_col * jax.lax.rsqrt(var + eps)
        t = bt_col - mean * s

        ones_hw = jnp.ones((1, HW), dtype=jnp.bfloat16)
        cst = bias_ref[...]
        for cc in range(C):
            wsum_c = jax.lax.dot_general(
                ones_hw, wb_ref[cc],
                dimension_numbers=(((1,), (1,)), ((), ())),
                preferred_element_type=jnp.float32)
            cst = cst + t[cc:cc + 1, :] * wsum_c

        acc = jnp.zeros(o_ref.shape, dtype=jnp.float32)
        for cc in range(C):
            acc = acc + p_ref[cc] * s[cc:cc + 1, :]
        o_ref[...] = acc + cst


def kernel(x, gamma, beta, weight, bias):
    B, C, H, W = x.shape
    HW = H * W
    F = C * HW
    K = weight.shape[0]

    xt = x.reshape(B, F).T

    cpg = 2 if C % 2 == 0 and C >= 2 else 1

    out = pl.pallas_call(
        functools.partial(_fused_bn_fc_kernel,
                          inv_n=1.0 / float(B * HW), eps=1e-5, cpg=cpg),
        out_shape=jax.ShapeDtypeStruct((B, K), jnp.float32),
        grid=(C // cpg,),
        in_specs=[pl.BlockSpec((cpg * HW, B), lambda c: (c, 0)),
                  pl.BlockSpec((1, C), lambda c: (0, 0)),
                  pl.BlockSpec((1, C), lambda c: (0, 0)),
                  pl.BlockSpec((K, F), lambda c: (0, 0)),
                  pl.BlockSpec((1, K), lambda c: (0, 0))],
        out_specs=pl.BlockSpec((B, K), lambda c: (0, 0)),
        scratch_shapes=[pltpu.VMEM((C, K, HW), jnp.bfloat16),
                        pltpu.VMEM((C, 1, B), jnp.float32),
                        pltpu.VMEM((C, 1, B), jnp.float32),
                        pltpu.VMEM((C, B, K), jnp.float32)],
        compiler_params=pltpu.CompilerParams(
            dimension_semantics=("arbitrary",),
            vmem_limit_bytes=56 * 1024 * 1024),
    )(xt, gamma.reshape(1, C), beta.reshape(1, C), weight, bias.reshape(1, K))
    return out


# speedup vs baseline: 3.1868x; 1.0848x over previous
import functools

import jax
import jax.numpy as jnp
from jax.experimental import pallas as pl
from jax.experimental.pallas import tpu as pltpu


def _fused_bn_fc_kernel(xt_ref, g_ref, bt_ref, w_ref, bias_ref,
                        o_ref,
                        wb_ref, sum_ref, sumsq_ref, p_ref,
                        *, inv_n, eps, cpg):
    j = pl.program_id(0)
    K, F = w_ref.shape
    C = g_ref.shape[1]
    HW = F // C

    @pl.when(j == 0)
    def _():
        for cc in range(C):
            wb_ref[cc] = w_ref[:, cc * HW:(cc + 1) * HW].astype(jnp.bfloat16)

    xf = xt_ref[...]
    xb = xf.astype(jnp.bfloat16)
    for i in range(cpg):
        c = cpg * j + i
        xfi = xf[i * HW:(i + 1) * HW]
        sum_ref[c] = jnp.sum(xfi, axis=0, keepdims=True)
        sumsq_ref[c] = jnp.sum(xfi * xfi, axis=0, keepdims=True)
        pc = jax.lax.dot_general(
            xb[i * HW:(i + 1) * HW], wb_ref[c],
            dimension_numbers=(((0,), (1,)), ((), ())),
            preferred_element_type=jnp.float32)
        p_ref[c] = pc

    @pl.when(j == pl.num_programs(0) - 1)
    def _():
        B = o_ref.shape[0]
        sums = sum_ref[...].reshape(C, B)
        sqs = sumsq_ref[...].reshape(C, B)
        mean = jnp.sum(sums, axis=1, keepdims=True) * inv_n
        var = jnp.sum(sqs, axis=1, keepdims=True) * inv_n - mean * mean
        var = jnp.maximum(var, 0.0)
        g_col = jnp.transpose(g_ref[...])
        bt_col = jnp.transpose(bt_ref[...])
        s = g_col * jax.lax.rsqrt(var + eps)
        t = bt_col - mean * s

        ones_hw = jnp.ones((1, HW), dtype=jnp.bfloat16)
        cst = bias_ref[...]
        for cc in range(C):
            wsum_c = jax.lax.dot_general(
                ones_hw, wb_ref[cc],
                dimension_numbers=(((1,), (1,)), ((), ())),
                preferred_element_type=jnp.float32)
            cst = cst + t[cc:cc + 1, :] * wsum_c

        acc = jnp.zeros(o_ref.shape, dtype=jnp.float32)
        for cc in range(C):
            acc = acc + p_ref[cc] * s[cc:cc + 1, :]
        o_ref[...] = acc + cst


def kernel(x, gamma, beta, weight, bias):
    B, C, H, W = x.shape
    HW = H * W
    F = C * HW
    K = weight.shape[0]

    xt = x.reshape(B, F).T

    cpg = 4 if C % 4 == 0 else (2 if C % 2 == 0 else 1)

    out = pl.pallas_call(
        functools.partial(_fused_bn_fc_kernel,
                          inv_n=1.0 / float(B * HW), eps=1e-5, cpg=cpg),
        out_shape=jax.ShapeDtypeStruct((B, K), jnp.float32),
        grid=(C // cpg,),
        in_specs=[pl.BlockSpec((cpg * HW, B), lambda c: (c, 0)),
                  pl.BlockSpec((1, C), lambda c: (0, 0)),
                  pl.BlockSpec((1, C), lambda c: (0, 0)),
                  pl.BlockSpec((K, F), lambda c: (0, 0)),
                  pl.BlockSpec((1, K), lambda c: (0, 0))],
        out_specs=pl.BlockSpec((B, K), lambda c: (0, 0)),
        scratch_shapes=[pltpu.VMEM((C, K, HW), jnp.bfloat16),
                        pltpu.VMEM((C, 1, B), jnp.float32),
                        pltpu.VMEM((C, 1, B), jnp.float32),
                        pltpu.VMEM((C, B, K), jnp.float32)],
        compiler_params=pltpu.CompilerParams(
            dimension_semantics=("arbitrary",),
            vmem_limit_bytes=56 * 1024 * 1024),
    )(xt, gamma.reshape(1, C), beta.reshape(1, C), weight, bias.reshape(1, K))
    return out
